# baseline (speedup 1.0000x reference)
import functools

import jax
import jax.numpy as jnp
from jax import lax
from jax.experimental import pallas as pl
from jax.experimental.pallas import tpu as pltpu
from jax.experimental.pallas import tpu_sc as plsc

_FEATURE_DIM = 256
_VOXEL = 0.4
_PC_RANGE = [-50.0, -50.0, -3.0, 50.0, 50.0, 3.0]
_GX = 250
_GY = 250
_CHUNK = 4096


def _layer_kernel(x_ref, w_ref, b_ref, h_ref, m0_ref, s1_ref, s2_ref):
    h = jnp.dot(x_ref[...], w_ref[...], preferred_element_type=jnp.float32)
    h = h + b_ref[...]
    h_ref[...] = h

    @pl.when(pl.program_id(0) == 0)
    def _():
        m0_ref[...] = jnp.mean(h, axis=0, keepdims=True)
        s1_ref[...] = jnp.zeros_like(s1_ref)
        s2_ref[...] = jnp.zeros_like(s2_ref)

    d = h - m0_ref[...]
    s1_ref[...] += jnp.sum(d, axis=0, keepdims=True)
    s2_ref[...] += jnp.sum(d * d, axis=0, keepdims=True)


def _layer_act_kernel(x_ref, a_ref, c_ref, w_ref, b_ref, h_ref, m0_ref,
                      s1_ref, s2_ref):
    act = jnp.maximum(x_ref[...] * a_ref[...] + c_ref[...], 0.0)
    h = jnp.dot(act, w_ref[...], preferred_element_type=jnp.float32)
    h = h + b_ref[...]
    h_ref[...] = h

    @pl.when(pl.program_id(0) == 0)
    def _():
        m0_ref[...] = jnp.mean(h, axis=0, keepdims=True)
        s1_ref[...] = jnp.zeros_like(s1_ref)
        s2_ref[...] = jnp.zeros_like(s2_ref)

    d = h - m0_ref[...]
    s1_ref[...] += jnp.sum(d, axis=0, keepdims=True)
    s2_ref[...] += jnp.sum(d * d, axis=0, keepdims=True)


def _final_kernel(x_ref, a_ref, c_ref, w_ref, b_ref, ft_ref):
    act = jnp.maximum(x_ref[...] * a_ref[...] + c_ref[...], 0.0)
    ft = lax.dot_general(w_ref[...], act, (((1,), (1,)), ((), ())),
                         preferred_element_type=jnp.float32)
    ft_ref[...] = ft + b_ref[...]


def _run_layer(x, w, b, width):
    m = x.shape[0]
    grid = (m // _CHUNK,)
    kin = x.shape[1]
    return pl.pallas_call(
        _layer_kernel,
        grid=grid,
        in_specs=[
            pl.BlockSpec((_CHUNK, kin), lambda i: (i, 0)),
            pl.BlockSpec((kin, width), lambda i: (0, 0)),
            pl.BlockSpec((1, width), lambda i: (0, 0)),
        ],
        out_specs=[
            pl.BlockSpec((_CHUNK, width), lambda i: (i, 0)),
            pl.BlockSpec((1, width), lambda i: (0, 0)),
            pl.BlockSpec((1, width), lambda i: (0, 0)),
            pl.BlockSpec((1, width), lambda i: (0, 0)),
        ],
        out_shape=[
            jax.ShapeDtypeStruct((m, width), jnp.float32),
            jax.ShapeDtypeStruct((1, width), jnp.float32),
            jax.ShapeDtypeStruct((1, width), jnp.float32),
            jax.ShapeDtypeStruct((1, width), jnp.float32),
        ],
    )(x, w, b)


def _run_layer_act(x, a, c, w, b, width):
    m = x.shape[0]
    grid = (m // _CHUNK,)
    kin = x.shape[1]
    return pl.pallas_call(
        _layer_act_kernel,
        grid=grid,
        in_specs=[
            pl.BlockSpec((_CHUNK, kin), lambda i: (i, 0)),
            pl.BlockSpec((1, kin), lambda i: (0, 0)),
            pl.BlockSpec((1, kin), lambda i: (0, 0)),
            pl.BlockSpec((kin, width), lambda i: (0, 0)),
            pl.BlockSpec((1, width), lambda i: (0, 0)),
        ],
        out_specs=[
            pl.BlockSpec((_CHUNK, width), lambda i: (i, 0)),
            pl.BlockSpec((1, width), lambda i: (0, 0)),
            pl.BlockSpec((1, width), lambda i: (0, 0)),
            pl.BlockSpec((1, width), lambda i: (0, 0)),
        ],
        out_shape=[
            jax.ShapeDtypeStruct((m, width), jnp.float32),
            jax.ShapeDtypeStruct((1, width), jnp.float32),
            jax.ShapeDtypeStruct((1, width), jnp.float32),
            jax.ShapeDtypeStruct((1, width), jnp.float32),
        ],
    )(x, a, c, w, b)


def _run_final(x, a, c, w, b):
    m = x.shape[0]
    kin = x.shape[1]
    grid = (m // _CHUNK,)
    return pl.pallas_call(
        _final_kernel,
        grid=grid,
        in_specs=[
            pl.BlockSpec((_CHUNK, kin), lambda i: (i, 0)),
            pl.BlockSpec((1, kin), lambda i: (0, 0)),
            pl.BlockSpec((1, kin), lambda i: (0, 0)),
            pl.BlockSpec((_FEATURE_DIM, kin), lambda i: (0, 0)),
            pl.BlockSpec((_FEATURE_DIM, 1), lambda i: (0, 0)),
        ],
        out_specs=pl.BlockSpec((_FEATURE_DIM, _CHUNK), lambda i: (0, i)),
        out_shape=jax.ShapeDtypeStruct((_FEATURE_DIM, m), jnp.float32),
    )(x, a, c, w, b)


def _bn_coeffs(m0, s1, s2, gamma, beta, m):
    mean = m0[0] + s1[0] / m
    var = s2[0] / m - (s1[0] / m) ** 2
    a = gamma * lax.rsqrt(var + 1e-5)
    c = beta - mean * a
    return a[None, :], c[None, :]


_NPTS = 65536
_SC_CHUNK = 1024
_PLANE = _GX * _GY
_PLANE_STRIDE = 62592
_ACC_LEN = 2 * _PLANE_STRIDE
_FEATS_PER_WORKER = _FEATURE_DIM // 32


def _scatter_max_kernel(featsT_hbm, flat_hbm, out_hbm, acc, idx_buf, val_buf):
    wid = lax.axis_index("s") * 2 + lax.axis_index("c")
    f_base = wid * _FEATS_PER_WORKER

    def feat_body(fi, carry):
        f = f_base + fi

        def zero_body(j, c):
            acc[pl.ds(j * 16, 16)] = jnp.zeros((16,), jnp.float32)
            return c

        lax.fori_loop(0, _ACC_LEN // 16, zero_body, 0)

        def chunk_body(ci, c):
            base = ci * _SC_CHUNK
            pltpu.sync_copy(flat_hbm.at[pl.ds(base, _SC_CHUNK)], idx_buf)
            pltpu.sync_copy(featsT_hbm.at[f, pl.ds(base, _SC_CHUNK)], val_buf)

            def grp_body(g, c2):
                idx = idx_buf[pl.ds(g * 16, 16)]
                val = val_buf[pl.ds(g * 16, 16)]
                cur = plsc.load_gather(acc, [idx])
                m = val > cur
                plsc.store_scatter(acc, [idx], val, mask=m)
                cur2 = plsc.load_gather(acc, [idx])
                pend = val > cur2

                def wcond(p):
                    return jnp.max(p.astype(jnp.int32)) > 0

                def wbody(p):
                    plsc.store_scatter(acc, [idx], val, mask=p)
                    c3 = plsc.load_gather(acc, [idx])
                    return val > c3

                lax.while_loop(wcond, wbody, pend)
                return c2

            lax.fori_loop(0, _SC_CHUNK // 16, grp_body, 0)
            return c

        lax.fori_loop(0, _NPTS // _SC_CHUNK, chunk_body, 0)

        off0 = pl.multiple_of(f * _PLANE_STRIDE, 128)
        off1 = pl.multiple_of((_FEATURE_DIM + f) * _PLANE_STRIDE, 128)
        pltpu.sync_copy(acc.at[pl.ds(0, _PLANE_STRIDE)],
                        out_hbm.at[pl.ds(off0, _PLANE_STRIDE)])
        pltpu.sync_copy(acc.at[pl.ds(_PLANE_STRIDE, _PLANE_STRIDE)],
                        out_hbm.at[pl.ds(off1, _PLANE_STRIDE)])
        return carry

    lax.fori_loop(0, _FEATS_PER_WORKER, feat_body, 0)


def _scatter_max(featsT, flat, batch):
    mesh = plsc.VectorSubcoreMesh(core_axis_name="c", subcore_axis_name="s")
    k = functools.partial(
        pl.kernel,
        mesh=mesh,
        out_type=jax.ShapeDtypeStruct((batch * _FEATURE_DIM * _PLANE_STRIDE,),
                                      jnp.float32),
        scratch_types=[
            pltpu.VMEM((_ACC_LEN,), jnp.float32),
            pltpu.VMEM((_SC_CHUNK,), jnp.int32),
            pltpu.VMEM((_SC_CHUNK,), jnp.float32),
        ],
        compiler_params=pltpu.CompilerParams(needs_layout_passes=False),
    )(_scatter_max_kernel)
    return k(featsT, flat)


def kernel(points, batch_indices, W1, b1, g1, be1, W2, b2, g2, be2, W3, b3):
    B, N, _ = points.shape
    m = B * N
    pts = points.reshape(m, -1)
    x = (pts[:, 0] - _PC_RANGE[0]) / _VOXEL
    y = (pts[:, 1] - _PC_RANGE[1]) / _VOXEL
    z = (pts[:, 2] - _PC_RANGE[2]) / _VOXEL
    pts = pts.at[:, 0].set(x).at[:, 1].set(y).at[:, 2].set(z)
    gx = jnp.clip(x.astype(jnp.int32), 0, _GX - 1)
    gy = jnp.clip(y.astype(jnp.int32), 0, _GY - 1)
    flat = batch_indices.astype(jnp.int32) * _PLANE_STRIDE + gx * _GY + gy

    pts8 = jnp.pad(pts, ((0, 0), (0, 1)))
    w1 = jnp.pad(W1, ((0, 0), (0, 1))).T
    h1, m0a, s1a, s2a = _run_layer(pts8, w1, b1[None, :], 64)
    a1, c1 = _bn_coeffs(m0a, s1a, s2a, g1, be1, m)

    h2, m0b, s1b, s2b = _run_layer_act(h1, a1, c1, W2.T, b2[None, :], 128)
    a2, c2 = _bn_coeffs(m0b, s1b, s2b, g2, be2, m)

    featsT = _run_final(h2, a2, c2, W3, b3[:, None])

    bev = _scatter_max(featsT, flat, B)
    bev = bev.reshape(B, _FEATURE_DIM, _PLANE_STRIDE)[:, :, :_PLANE]
    return bev.reshape(B, _FEATURE_DIM, _GX, _GY)


# speedup vs baseline: 2.1762x; 2.1762x over previous
import functools

import jax
import jax.numpy as jnp
from jax import lax
from jax.experimental import pallas as pl
from jax.experimental.pallas import tpu as pltpu
from jax.experimental.pallas import tpu_sc as plsc

_FEATURE_DIM = 256
_VOXEL = 0.4
_PC_RANGE = [-50.0, -50.0, -3.0, 50.0, 50.0, 3.0]
_GX = 250
_GY = 250
_CHUNK = 4096


def _layer_kernel(x_ref, w_ref, b_ref, h_ref, m0_ref, s1_ref, s2_ref):
    h = jnp.dot(x_ref[...], w_ref[...], preferred_element_type=jnp.float32)
    h = h + b_ref[...]
    h_ref[...] = h

    @pl.when(pl.program_id(0) == 0)
    def _():
        m0_ref[...] = jnp.mean(h, axis=0, keepdims=True)
        s1_ref[...] = jnp.zeros_like(s1_ref)
        s2_ref[...] = jnp.zeros_like(s2_ref)

    d = h - m0_ref[...]
    s1_ref[...] += jnp.sum(d, axis=0, keepdims=True)
    s2_ref[...] += jnp.sum(d * d, axis=0, keepdims=True)


def _layer_act_kernel(x_ref, a_ref, c_ref, w_ref, b_ref, h_ref, m0_ref,
                      s1_ref, s2_ref):
    act = jnp.maximum(x_ref[...] * a_ref[...] + c_ref[...], 0.0)
    h = jnp.dot(act, w_ref[...], preferred_element_type=jnp.float32)
    h = h + b_ref[...]
    h_ref[...] = h

    @pl.when(pl.program_id(0) == 0)
    def _():
        m0_ref[...] = jnp.mean(h, axis=0, keepdims=True)
        s1_ref[...] = jnp.zeros_like(s1_ref)
        s2_ref[...] = jnp.zeros_like(s2_ref)

    d = h - m0_ref[...]
    s1_ref[...] += jnp.sum(d, axis=0, keepdims=True)
    s2_ref[...] += jnp.sum(d * d, axis=0, keepdims=True)


def _final_kernel(x_ref, a_ref, c_ref, w_ref, b_ref, ft_ref):
    act = jnp.maximum(x_ref[...] * a_ref[...] + c_ref[...], 0.0)
    ft = lax.dot_general(w_ref[...], act, (((1,), (1,)), ((), ())),
                         preferred_element_type=jnp.float32)
    ft_ref[...] = ft + b_ref[...]


def _run_layer(x, w, b, width):
    m = x.shape[0]
    grid = (m // _CHUNK,)
    kin = x.shape[1]
    return pl.pallas_call(
        _layer_kernel,
        grid=grid,
        in_specs=[
            pl.BlockSpec((_CHUNK, kin), lambda i: (i, 0)),
            pl.BlockSpec((kin, width), lambda i: (0, 0)),
            pl.BlockSpec((1, width), lambda i: (0, 0)),
        ],
        out_specs=[
            pl.BlockSpec((_CHUNK, width), lambda i: (i, 0)),
            pl.BlockSpec((1, width), lambda i: (0, 0)),
            pl.BlockSpec((1, width), lambda i: (0, 0)),
            pl.BlockSpec((1, width), lambda i: (0, 0)),
        ],
        out_shape=[
            jax.ShapeDtypeStruct((m, width), jnp.float32),
            jax.ShapeDtypeStruct((1, width), jnp.float32),
            jax.ShapeDtypeStruct((1, width), jnp.float32),
            jax.ShapeDtypeStruct((1, width), jnp.float32),
        ],
    )(x, w, b)


def _run_layer_act(x, a, c, w, b, width):
    m = x.shape[0]
    grid = (m // _CHUNK,)
    kin = x.shape[1]
    return pl.pallas_call(
        _layer_act_kernel,
        grid=grid,
        in_specs=[
            pl.BlockSpec((_CHUNK, kin), lambda i: (i, 0)),
            pl.BlockSpec((1, kin), lambda i: (0, 0)),
            pl.BlockSpec((1, kin), lambda i: (0, 0)),
            pl.BlockSpec((kin, width), lambda i: (0, 0)),
            pl.BlockSpec((1, width), lambda i: (0, 0)),
        ],
        out_specs=[
            pl.BlockSpec((_CHUNK, width), lambda i: (i, 0)),
            pl.BlockSpec((1, width), lambda i: (0, 0)),
            pl.BlockSpec((1, width), lambda i: (0, 0)),
            pl.BlockSpec((1, width), lambda i: (0, 0)),
        ],
        out_shape=[
            jax.ShapeDtypeStruct((m, width), jnp.float32),
            jax.ShapeDtypeStruct((1, width), jnp.float32),
            jax.ShapeDtypeStruct((1, width), jnp.float32),
            jax.ShapeDtypeStruct((1, width), jnp.float32),
        ],
    )(x, a, c, w, b)


def _run_final(x, a, c, w, b):
    m = x.shape[0]
    kin = x.shape[1]
    grid = (m // _CHUNK,)
    return pl.pallas_call(
        _final_kernel,
        grid=grid,
        in_specs=[
            pl.BlockSpec((_CHUNK, kin), lambda i: (i, 0)),
            pl.BlockSpec((1, kin), lambda i: (0, 0)),
            pl.BlockSpec((1, kin), lambda i: (0, 0)),
            pl.BlockSpec((_FEATURE_DIM, kin), lambda i: (0, 0)),
            pl.BlockSpec((_FEATURE_DIM, 1), lambda i: (0, 0)),
        ],
        out_specs=pl.BlockSpec((_FEATURE_DIM, _CHUNK), lambda i: (0, i)),
        out_shape=jax.ShapeDtypeStruct((_FEATURE_DIM, m), jnp.float32),
    )(x, a, c, w, b)


def _bn_coeffs(m0, s1, s2, gamma, beta, m):
    mean = m0[0] + s1[0] / m
    var = s2[0] / m - (s1[0] / m) ** 2
    a = gamma * lax.rsqrt(var + 1e-5)
    c = beta - mean * a
    return a[None, :], c[None, :]


_NPTS = 65536
_SC_CHUNK = 1024
_PLANE = _GX * _GY
_PLANE_STRIDE = 62592
_ACC_LEN = 2 * _PLANE_STRIDE
_FEATS_PER_WORKER = _FEATURE_DIM // 32


def _scatter_max_kernel(featsT_hbm, flat_hbm, out_hbm, acc,
                        idx0, idx1, val0, val1, sem0, sem1):
    wid = lax.axis_index("s") * 2 + lax.axis_index("c")
    f_base = wid * _FEATS_PER_WORKER
    n_chunks = _NPTS // _SC_CHUNK
    pend0 = jnp.zeros((16,), jnp.bool_)

    def feat_body(fi, carry):
        f = f_base + fi

        zero = jnp.zeros((16,), jnp.float32)

        def zero_body(j, c):
            for u in range(8):
                acc[pl.ds(j * 128 + u * 16, 16)] = zero
            return c

        lax.fori_loop(0, _ACC_LEN // 128, zero_body, 0)

        def fire(ci, ib, vb, sem):
            base = ci * _SC_CHUNK
            pltpu.async_copy(flat_hbm.at[pl.ds(base, _SC_CHUNK)], ib, sem)
            pltpu.async_copy(featsT_hbm.at[f, pl.ds(base, _SC_CHUNK)],
                             vb, sem)

        def drain(ci, ib, vb, sem):
            base = ci * _SC_CHUNK
            pltpu.make_async_copy(
                flat_hbm.at[pl.ds(base, _SC_CHUNK)], ib, sem).wait()
            pltpu.make_async_copy(
                featsT_hbm.at[f, pl.ds(base, _SC_CHUNK)], vb, sem).wait()

        def process(ib, vb):
            def one_pass(k, pend):
                for u in range(8):
                    g = k * 8 + u
                    idx = ib[pl.ds(g * 16, 16)]
                    val = vb[pl.ds(g * 16, 16)]
                    cur = plsc.load_gather(acc, [idx])
                    plsc.store_scatter(acc, [idx], val, mask=val > cur)
                    cur2 = plsc.load_gather(acc, [idx])
                    pend = jnp.logical_or(pend, val > cur2)
                return pend

            pend = lax.fori_loop(0, _SC_CHUNK // 128, one_pass, pend0)

            def wcond(p):
                return jnp.max(p.astype(jnp.int32)) > 0

            def wbody(p):
                return lax.fori_loop(0, _SC_CHUNK // 128, one_pass, pend0)

            lax.while_loop(wcond, wbody, pend)

        fire(0, idx0, val0, sem0)

        def dchunk(k, c):
            c0 = 2 * k
            fire(c0 + 1, idx1, val1, sem1)
            drain(c0, idx0, val0, sem0)
            process(idx0, val0)
            nxt = jnp.minimum(c0 + 2, n_chunks - 2)
            fire(nxt, idx0, val0, sem0)
            drain(c0 + 1, idx1, val1, sem1)
            process(idx1, val1)
            return c

        lax.fori_loop(0, n_chunks // 2, dchunk, 0)
        drain(0, idx0, val0, sem0)

        off0 = pl.multiple_of(f * _PLANE_STRIDE, 128)
        off1 = pl.multiple_of((_FEATURE_DIM + f) * _PLANE_STRIDE, 128)
        pltpu.sync_copy(acc.at[pl.ds(0, _PLANE_STRIDE)],
                        out_hbm.at[pl.ds(off0, _PLANE_STRIDE)])
        pltpu.sync_copy(acc.at[pl.ds(_PLANE_STRIDE, _PLANE_STRIDE)],
                        out_hbm.at[pl.ds(off1, _PLANE_STRIDE)])
        return carry

    lax.fori_loop(0, _FEATS_PER_WORKER, feat_body, 0)


def _scatter_max(featsT, flat, batch):
    mesh = plsc.VectorSubcoreMesh(core_axis_name="c", subcore_axis_name="s")
    k = functools.partial(
        pl.kernel,
        mesh=mesh,
        out_type=jax.ShapeDtypeStruct((batch * _FEATURE_DIM * _PLANE_STRIDE,),
                                      jnp.float32),
        scratch_types=[
            pltpu.VMEM((_ACC_LEN,), jnp.float32),
            pltpu.VMEM((_SC_CHUNK,), jnp.int32),
            pltpu.VMEM((_SC_CHUNK,), jnp.int32),
            pltpu.VMEM((_SC_CHUNK,), jnp.float32),
            pltpu.VMEM((_SC_CHUNK,), jnp.float32),
            pltpu.SemaphoreType.DMA,
            pltpu.SemaphoreType.DMA,
        ],
        compiler_params=pltpu.CompilerParams(needs_layout_passes=False),
    )(_scatter_max_kernel)
    return k(featsT, flat)


def kernel(points, batch_indices, W1, b1, g1, be1, W2, b2, g2, be2, W3, b3):
    B, N, _ = points.shape
    m = B * N
    pts = points.reshape(m, -1)
    x = (pts[:, 0] - _PC_RANGE[0]) / _VOXEL
    y = (pts[:, 1] - _PC_RANGE[1]) / _VOXEL
    z = (pts[:, 2] - _PC_RANGE[2]) / _VOXEL
    pts = pts.at[:, 0].set(x).at[:, 1].set(y).at[:, 2].set(z)
    gx = jnp.clip(x.astype(jnp.int32), 0, _GX - 1)
    gy = jnp.clip(y.astype(jnp.int32), 0, _GY - 1)
    flat = batch_indices.astype(jnp.int32) * _PLANE_STRIDE + gx * _GY + gy

    pts8 = jnp.pad(pts, ((0, 0), (0, 1)))
    w1 = jnp.pad(W1, ((0, 0), (0, 1))).T
    h1, m0a, s1a, s2a = _run_layer(pts8, w1, b1[None, :], 64)
    a1, c1 = _bn_coeffs(m0a, s1a, s2a, g1, be1, m)

    h2, m0b, s1b, s2b = _run_layer_act(h1, a1, c1, W2.T, b2[None, :], 128)
    a2, c2 = _bn_coeffs(m0b, s1b, s2b, g2, be2, m)

    featsT = _run_final(h2, a2, c2, W3, b3[:, None])

    bev = _scatter_max(featsT, flat, B)
    bev = bev.reshape(B, _FEATURE_DIM, _PLANE_STRIDE)[:, :, :_PLANE]
    return bev.reshape(B, _FEATURE_DIM, _GX, _GY)
